# baseline (speedup 1.0000x reference)
import functools
import math

import jax
import jax.numpy as jnp
from jax import lax
from jax.experimental import pallas as pl
from jax.experimental.pallas import tpu as pltpu
from jax.experimental.pallas import tpu_sc as plsc

EMB = 64
SCALE = math.sqrt(EMB)

NUM_WORKERS = 32
CHUNK = 512


def _make_sc_lookup(B, D, b_per_w, n_chunks):
    mesh = plsc.VectorSubcoreMesh(core_axis_name="c", subcore_axis_name="s")

    @functools.partial(
        pl.kernel,
        mesh=mesh,
        out_type=jax.ShapeDtypeStruct((B, D), jnp.float32),
        scratch_types=[
            pltpu.VMEM((b_per_w,), jnp.int32),
            pltpu.VMEM((CHUNK, D), jnp.float32),
            pltpu.SemaphoreType.DMA,
        ],
        compiler_params=pltpu.CompilerParams(use_tc_tiling_on_sc=False),
    )
    def lookup(table_hbm, idx_hbm, out_hbm, idx_v, rows_v, sem):
        wid = lax.axis_index("s") * 2 + lax.axis_index("c")
        base = wid * b_per_w
        pltpu.sync_copy(idx_hbm.at[pl.ds(base, b_per_w)], idx_v)

        def chunk_body(g, carry):
            pltpu.async_copy(
                table_hbm.at[idx_v.at[pl.ds(g * CHUNK, CHUNK)]], rows_v, sem
            ).wait()

            def scale_row(i, carry2):
                for j in range(D // 16):
                    sl = pl.ds(j * 16, 16)
                    rows_v[i, sl] = rows_v[i, sl] * SCALE
                return carry2

            lax.fori_loop(0, CHUNK, scale_row, 0, unroll=2)
            pltpu.sync_copy(rows_v, out_hbm.at[pl.ds(base + g * CHUNK, CHUNK)])
            return carry

        lax.fori_loop(0, n_chunks, chunk_body, 0)

    return lookup


def kernel(token_sequences, table):
    Bseq, L = token_sequences.shape
    V, D = table.shape
    B = Bseq * L
    b_per_w = B // NUM_WORKERS
    n_chunks = b_per_w // CHUNK
    idx_flat = token_sequences.reshape(B)
    out = _make_sc_lookup(B, D, b_per_w, n_chunks)(table, idx_flat)
    return out.reshape(Bseq, L, D)


# speedup vs baseline: 1.0692x; 1.0692x over previous
import functools
import math

import jax
import jax.numpy as jnp
from jax import lax
from jax.experimental import pallas as pl
from jax.experimental.pallas import tpu as pltpu
from jax.experimental.pallas import tpu_sc as plsc

EMB = 64
SCALE = math.sqrt(EMB)

NUM_WORKERS = 32
CHUNK = 512


def _make_sc_lookup(B, D, b_per_w, n_chunks):
    mesh = plsc.VectorSubcoreMesh(core_axis_name="c", subcore_axis_name="s")
    nvec = D // 16

    @functools.partial(
        pl.kernel,
        mesh=mesh,
        out_type=jax.ShapeDtypeStruct((B, D), jnp.float32),
        scratch_types=[
            pltpu.VMEM((b_per_w,), jnp.int32),
            pltpu.VMEM((CHUNK, D), jnp.float32),
            pltpu.VMEM((CHUNK, D), jnp.float32),
            pltpu.SemaphoreType.DMA,
            pltpu.SemaphoreType.DMA,
            pltpu.SemaphoreType.DMA,
            pltpu.SemaphoreType.DMA,
        ],
        compiler_params=pltpu.CompilerParams(use_tc_tiling_on_sc=False),
    )
    def lookup(table_hbm, idx_hbm, out_hbm, idx_v, rows0, rows1,
               gsem0, gsem1, ssem0, ssem1):
        rows = (rows0, rows1)
        gsem = (gsem0, gsem1)
        ssem = (ssem0, ssem1)
        wid = lax.axis_index("s") * 2 + lax.axis_index("c")
        base = wid * b_per_w
        pltpu.sync_copy(idx_hbm.at[pl.ds(base, b_per_w)], idx_v)

        def gather(g, b):
            return pltpu.async_copy(
                table_hbm.at[idx_v.at[pl.ds(g * CHUNK, CHUNK)]], rows[b],
                gsem[b])

        def scatter(g, b):
            return pltpu.async_copy(
                rows[b], out_hbm.at[pl.ds(base + g * CHUNK, CHUNK)], ssem[b])

        def wait_scatter(g, b):
            pltpu.make_async_copy(
                rows[b], out_hbm.at[pl.ds(base + g * CHUNK, CHUNK)],
                ssem[b]).wait()

        def wait_gather(g, b):
            pltpu.make_async_copy(
                table_hbm.at[idx_v.at[pl.ds(g * CHUNK, CHUNK)]], rows[b],
                gsem[b]).wait()

        def scale(b):
            buf = rows[b]

            @plsc.parallel_loop(0, CHUNK, unroll=4)
            def _(i):
                for j in range(nvec):
                    sl = pl.ds(j * 16, 16)
                    buf[i, sl] = buf[i, sl] * SCALE

        def step(g, b, first, last):
            other = 1 - b
            if not first:
                wait_scatter(g - 1, other)
            if not last:
                gather(g + 1, other)
            wait_gather(g, b)
            scale(b)
            scatter(g, b)

        gather(0, 0)
        step(0, 0, first=True, last=False)
        step(1, 1, first=False, last=False)

        def outer(t, carry):
            g = 2 * t
            step(g, 0, first=False, last=False)
            step(g + 1, 1, first=False, last=False)
            return carry

        lax.fori_loop(1, n_chunks // 2 - 1, outer, 0)
        step(n_chunks - 2, 0, first=False, last=False)
        step(n_chunks - 1, 1, first=False, last=True)
        wait_scatter(n_chunks - 1, 1)

    return lookup


def kernel(token_sequences, table):
    Bseq, L = token_sequences.shape
    V, D = table.shape
    B = Bseq * L
    b_per_w = B // NUM_WORKERS
    n_chunks = b_per_w // CHUNK
    idx_flat = token_sequences.reshape(B)
    out = _make_sc_lookup(B, D, b_per_w, n_chunks)(table, idx_flat)
    return out.reshape(Bseq, L, D)
